# baseline (speedup 1.0000x reference)
import functools
import math

import jax
import jax.numpy as jnp
from jax import lax
from jax.experimental import pallas as pl
from jax.experimental.pallas import tpu as pltpu
from jax.experimental.pallas import tpu_sc as plsc

NUM_BUCKETS = 32
MAX_DISTANCE = 128
HEADS = 16
SEQ_I = 2048
SEQ_J = 2048

EXT_W = 4224
NP = 16
NQ = 32
NA = SEQ_I // 8
NCOLT = SEQ_J // 128

NC = 2
NS = 16

QSPAN = 23


def _tab5_tc_kernel(tbl_ref, out_ref, ext_ref):
    h = pl.program_id(0)
    max_exact = NUM_BUCKETS // 2
    d = (
        lax.broadcasted_iota(jnp.int32, (8, EXT_W), 1)
        + 7
        - lax.broadcasted_iota(jnp.int32, (8, EXT_W), 0)
    )
    n = jnp.maximum((SEQ_I - 1) - d, 0)
    nf = jnp.maximum(n, 1).astype(jnp.float32)
    val_large = max_exact + (
        jnp.log(nf / max_exact)
        / math.log(MAX_DISTANCE / max_exact)
        * (NUM_BUCKETS - max_exact)
    ).astype(jnp.int32)
    val_large = jnp.minimum(val_large, NUM_BUCKETS - 1)
    bucket = jnp.where(n < max_exact, n, val_large)
    acc = jnp.zeros((8, EXT_W), jnp.float32)
    for b in range(NUM_BUCKETS):
        acc = jnp.where(bucket == b, tbl_ref[h, b], acc)
    ext_ref[...] = acc
    for p in range(NP):
        sh = 120 - 8 * p
        slab = lax.slice(acc, (0, sh), (8, sh + NQ * 128))
        for q in range(NQ):
            out_ref[0, p, q, :, :] = slab[:, 128 * q : 128 * (q + 1)]


def _sc_tiled_writer(tab5_hbm, out_hbm, buf, ssem, wsem):
    c = lax.axis_index("c")
    s = lax.axis_index("s")
    wid = s * NC + c
    h = wid // 2
    half = wid % 2
    alo = half * 8
    qbase = 8 - alo

    def stage(p):
        return pltpu.async_copy(
            tab5_hbm.at[h, p, pl.ds(qbase, QSPAN), :, :],
            buf.at[p % 2],
            ssem,
        )

    stage_descs = [stage(0)]
    for p in range(NP):
        stage_descs[p].wait()
        if p + 1 < NP:
            stage_descs.append(stage(p + 1))
        wdescs = []
        for ar in range(8):
            alpha = alo + ar
            grp = 16 * alpha + p
            wdescs.append(
                pltpu.async_copy(
                    buf.at[p % 2, pl.ds(7 - ar, 16), :, :],
                    out_hbm.at[h, grp, :, :, :],
                    wsem,
                )
            )
        for d in wdescs:
            d.wait()


@jax.jit
def _impl(table):
    tab5 = pl.pallas_call(
        _tab5_tc_kernel,
        grid=(HEADS,),
        in_specs=[pl.BlockSpec(memory_space=pltpu.SMEM)],
        out_specs=pl.BlockSpec(
            (1, NP, NQ, 8, 128), lambda h: (h, 0, 0, 0, 0)
        ),
        out_shape=jax.ShapeDtypeStruct((HEADS, NP, NQ, 8, 128), jnp.float32),
        scratch_shapes=[pltpu.VMEM((8, EXT_W), jnp.float32)],
    )(table.T)

    sc_materialize = functools.partial(
        pl.kernel,
        mesh=plsc.VectorSubcoreMesh(core_axis_name="c", subcore_axis_name="s"),
        out_type=jax.ShapeDtypeStruct((HEADS, NA, NCOLT, 8, 128), jnp.float32),
        scratch_types=[
            pltpu.VMEM((2, QSPAN, 8, 128), jnp.float32),
            pltpu.SemaphoreType.DMA,
            pltpu.SemaphoreType.DMA,
        ],
    )(_sc_tiled_writer)
    out5 = sc_materialize(tab5)
    return jnp.transpose(out5, (0, 1, 3, 2, 4)).reshape(HEADS, SEQ_I, SEQ_J)


def kernel(i, j, relative_attention_bias):
    del i, j
    return _impl(relative_attention_bias)


# speedup vs baseline: 1.0021x; 1.0021x over previous
import functools
import math

import jax
import jax.numpy as jnp
from jax import lax
from jax.experimental import pallas as pl
from jax.experimental.pallas import tpu as pltpu
from jax.experimental.pallas import tpu_sc as plsc

NUM_BUCKETS = 32
MAX_DISTANCE = 128
HEADS = 16
SEQ_I = 2048
SEQ_J = 2048

EXT_W = 4224
NP = 16
NQ = 32
NA = SEQ_I // 8
NCOLT = SEQ_J // 128

NC = 2
NS = 16

QSPAN = 23


def _tab5_tc_kernel(tbl_ref, out_ref):
    h = pl.program_id(0)
    max_exact = NUM_BUCKETS // 2
    d = (
        lax.broadcasted_iota(jnp.int32, (8, EXT_W), 1)
        + 7
        - lax.broadcasted_iota(jnp.int32, (8, EXT_W), 0)
    )
    n = jnp.maximum((SEQ_I - 1) - d, 0)
    nf = jnp.maximum(n, 1).astype(jnp.float32)
    val_large = max_exact + (
        jnp.log(nf / max_exact)
        / math.log(MAX_DISTANCE / max_exact)
        * (NUM_BUCKETS - max_exact)
    ).astype(jnp.int32)
    val_large = jnp.minimum(val_large, NUM_BUCKETS - 1)
    bucket = jnp.where(n < max_exact, n, val_large)
    acc = jnp.zeros((8, EXT_W), jnp.float32)
    for b in range(NUM_BUCKETS):
        acc = jnp.where(bucket == b, tbl_ref[h, b], acc)
    for p in range(NP):
        sh = 120 - 8 * p
        slab = lax.slice(acc, (0, sh), (8, sh + NQ * 128))
        for q in range(NQ):
            out_ref[0, p, q, :, :] = slab[:, 128 * q : 128 * (q + 1)]


def _sc_tiled_writer(tab5_hbm, out_hbm, buf, ssem, wsem):
    c = lax.axis_index("c")
    s = lax.axis_index("s")
    wid = s * NC + c
    h = wid // 2
    half = wid % 2
    alo = half * 8
    qbase = 8 - alo

    def stage(p):
        return pltpu.async_copy(
            tab5_hbm.at[h, p, pl.ds(qbase, QSPAN), :, :],
            buf.at[p % 2],
            ssem,
        )

    stage_descs = [stage(0)]
    for p in range(NP):
        stage_descs[p].wait()
        if p + 1 < NP:
            stage_descs.append(stage(p + 1))
        wdescs = []
        for ar in range(8):
            alpha = alo + ar
            grp = 16 * alpha + p
            wdescs.append(
                pltpu.async_copy(
                    buf.at[p % 2, pl.ds(7 - ar, 16), :, :],
                    out_hbm.at[h, grp, :, :, :],
                    wsem,
                )
            )
        for d in wdescs:
            d.wait()


@jax.jit
def _impl(table):
    tab5 = pl.pallas_call(
        _tab5_tc_kernel,
        grid=(HEADS,),
        in_specs=[pl.BlockSpec(memory_space=pltpu.SMEM)],
        out_specs=pl.BlockSpec(
            (1, NP, NQ, 8, 128), lambda h: (h, 0, 0, 0, 0)
        ),
        out_shape=jax.ShapeDtypeStruct((HEADS, NP, NQ, 8, 128), jnp.float32),
    )(table.T)

    sc_materialize = functools.partial(
        pl.kernel,
        mesh=plsc.VectorSubcoreMesh(core_axis_name="c", subcore_axis_name="s"),
        out_type=jax.ShapeDtypeStruct((HEADS, NA, NCOLT, 8, 128), jnp.float32),
        scratch_types=[
            pltpu.VMEM((2, QSPAN, 8, 128), jnp.float32),
            pltpu.SemaphoreType.DMA,
            pltpu.SemaphoreType.DMA,
        ],
    )(_sc_tiled_writer)
    out5 = sc_materialize(tab5)
    return jnp.transpose(out5, (0, 1, 3, 2, 4)).reshape(HEADS, SEQ_I, SEQ_J)


def kernel(i, j, relative_attention_bias):
    del i, j
    return _impl(relative_attention_bias)
